# baseline (speedup 1.0000x reference)
import functools

import jax
import jax.numpy as jnp
from jax import lax
from jax.experimental import pallas as pl
from jax.experimental.pallas import tpu as pltpu
from jax.experimental.pallas import tpu_sc as plsc

L = 16


def _take16(v, idx):
    return lax.gather(
        v,
        idx[:, None],
        lax.GatherDimensionNumbers(
            offset_dims=(), collapsed_slice_dims=(0,), start_index_map=(0,)
        ),
        slice_sizes=(1,),
        mode=lax.GatherScatterMode.PROMISE_IN_BOUNDS,
    )


def _mean_color_sc(color, order, n_bins_pad):
    n_bands, n_rows, t_len = color.shape
    info = plsc.get_sparse_core_info()
    nw = info.num_cores * info.num_subcores
    rows_per_w = n_rows // nw
    mesh = plsc.VectorSubcoreMesh(core_axis_name="c", subcore_axis_name="s")

    in_t = [
        pltpu.VMEM((t_len,), jnp.float32),
        pltpu.VMEM((t_len,), jnp.float32),
        pltpu.VMEM((t_len,), jnp.int32),
        pltpu.VMEM((t_len,), jnp.int32),
    ]

    @functools.partial(
        pl.kernel,
        mesh=mesh,
        out_type=jax.ShapeDtypeStruct((n_rows, n_bins_pad), jnp.float32),
        compiler_params=pltpu.CompilerParams(
            needs_layout_passes=False, use_tc_tiling_on_sc=True
        ),
        scratch_types=in_t + in_t + [
            pltpu.VMEM((n_bins_pad,), jnp.float32),
            pltpu.VMEM((n_bins_pad,), jnp.float32),
            pltpu.VMEM((n_bins_pad,), jnp.float32),
            pltpu.VMEM((n_bins_pad,), jnp.float32),
            pltpu.SemaphoreType.DMA,
            pltpu.SemaphoreType.DMA,
        ],
    )
    def k(color_hbm, order_hbm, out_hbm,
          ca0, ca1, oa0, oa1, cb0, cb1, ob0, ob1,
          b0, b1, orow_a, orow_b, isem, osem):
        wid = lax.axis_index("s") * info.num_cores + lax.axis_index("c")
        row0 = wid * rows_per_w
        iota = lax.iota(jnp.int32, L)
        zeros = jnp.zeros((L,), jnp.float32)
        last_lane = jnp.full((L,), L - 1, jnp.int32)

        def issue_in(r, c0, c1, o0, o1):
            pltpu.async_copy(color_hbm.at[0, r], c0, isem)
            pltpu.async_copy(color_hbm.at[1, r], c1, isem)
            pltpu.async_copy(order_hbm.at[0, r], o0, isem)
            pltpu.async_copy(order_hbm.at[1, r], o1, isem)

        def wait_in(r, c0, c1, o0, o1):
            pltpu.make_async_copy(color_hbm.at[0, r], c0, isem).wait()
            pltpu.make_async_copy(color_hbm.at[1, r], c1, isem).wait()
            pltpu.make_async_copy(order_hbm.at[0, r], o0, isem).wait()
            pltpu.make_async_copy(order_hbm.at[1, r], o1, isem).wait()

        @plsc.parallel_loop(0, n_bins_pad, step=L, unroll=4)
        def _(i):
            s = pl.ds(i, L)
            b0[s] = zeros
            b1[s] = zeros

        issue_in(row0, ca0, ca1, oa0, oa1)

        def process_row(r, c0, c1, o0, o1, orow):
            @plsc.parallel_loop(0, t_len, step=L, unroll=8)
            def _(i):
                s = pl.ds(i, L)
                plsc.addupdate_scatter(b0, [o0[s]], c0[s])
                plsc.addupdate_scatter(b1, [o1[s]], c1[s])

            def ff_chunk(s, cy0, cy1):
                v0 = b0[s]
                v1 = b1[s]
                g0 = _take16(v0, plsc.cummax(iota, mask=v0 != 0.0))
                g1 = _take16(v1, plsc.cummax(iota, mask=v1 != 0.0))
                h0 = _take16(g0, last_lane)
                h1 = _take16(g1, last_lane)
                f0 = jnp.where(g0 != 0.0, g0, cy0)
                f1 = jnp.where(g1 != 0.0, g1, cy1)
                b0[s] = zeros
                b1[s] = zeros
                orow[s] = f0 - f1
                return (
                    jnp.where(h0 != 0.0, h0, cy0),
                    jnp.where(h1 != 0.0, h1, cy1),
                )

            @plsc.parallel_loop(
                0, n_bins_pad, step=L, unroll=8, carry=(zeros, zeros)
            )
            def _(i, carry):
                cy0, cy1 = carry
                return ff_chunk(pl.ds(i, L), cy0, cy1)
            pltpu.async_copy(orow, out_hbm.at[r], osem)

        def pair_body(rp, _):
            ra = row0 + 2 * rp
            rb = ra + 1
            wait_in(ra, ca0, ca1, oa0, oa1)
            issue_in(rb, cb0, cb1, ob0, ob1)

            @pl.when(rp > 0)
            def _():
                pltpu.make_async_copy(orow_a, out_hbm.at[ra - 2], osem).wait()

            process_row(ra, ca0, ca1, oa0, oa1, orow_a)

            wait_in(rb, cb0, cb1, ob0, ob1)

            @pl.when(rp + 1 < rows_per_w // 2)
            def _():
                issue_in(rb + 1, ca0, ca1, oa0, oa1)

            @pl.when(rp > 0)
            def _():
                pltpu.make_async_copy(orow_b, out_hbm.at[rb - 2], osem).wait()

            process_row(rb, cb0, cb1, ob0, ob1, orow_b)
            return 0

        lax.fori_loop(0, rows_per_w // 2, pair_body, 0)

        last = row0 + rows_per_w - 1
        pltpu.make_async_copy(orow_a, out_hbm.at[last - 1], osem).wait()
        pltpu.make_async_copy(orow_b, out_hbm.at[last], osem).wait()

    return k(color, order)


def kernel(color, Ns, order):
    n_bands = color.shape[0]
    bsz = color.shape[1]
    ns_bands, ns_rows = Ns.shape
    n_bins = ns_rows * ns_bands * (ns_bands - 1) // 2 + ns_bands * (ns_rows - 1)
    n_bins_pad = (n_bins + 6 * L - 1) // (6 * L) * (6 * L)

    out = _mean_color_sc(color, order.astype(jnp.int32), n_bins_pad)
    return out[:, :n_bins].reshape(bsz, n_bins, 1)


# speedup vs baseline: 1.9963x; 1.9963x over previous
import functools

import jax
import jax.numpy as jnp
from jax import lax
from jax.experimental import pallas as pl
from jax.experimental.pallas import tpu as pltpu
from jax.experimental.pallas import tpu_sc as plsc

L = 16


def _take16(v, idx):
    return lax.gather(
        v,
        idx[:, None],
        lax.GatherDimensionNumbers(
            offset_dims=(), collapsed_slice_dims=(0,), start_index_map=(0,)
        ),
        slice_sizes=(1,),
        mode=lax.GatherScatterMode.PROMISE_IN_BOUNDS,
    )


def _mean_color_sc(color, order, n_bins_pad):
    n_bands, n_rows, t_len = color.shape
    info = plsc.get_sparse_core_info()
    nw = info.num_cores * info.num_subcores
    rows_per_w = n_rows // nw
    mesh = plsc.VectorSubcoreMesh(core_axis_name="c", subcore_axis_name="s")

    in_t = [
        pltpu.VMEM((t_len,), jnp.float32),
        pltpu.VMEM((t_len,), jnp.float32),
        pltpu.VMEM((t_len,), jnp.int32),
        pltpu.VMEM((t_len,), jnp.int32),
    ]

    @functools.partial(
        pl.kernel,
        mesh=mesh,
        out_type=jax.ShapeDtypeStruct((n_rows, n_bins_pad), jnp.float32),
        compiler_params=pltpu.CompilerParams(
            needs_layout_passes=False, use_tc_tiling_on_sc=True
        ),
        scratch_types=in_t + in_t + [
            pltpu.VMEM((n_bins_pad,), jnp.float32),
            pltpu.VMEM((n_bins_pad,), jnp.float32),
            pltpu.VMEM((n_bins_pad,), jnp.float32),
            pltpu.VMEM((n_bins_pad,), jnp.float32),
            pltpu.SemaphoreType.DMA,
            pltpu.SemaphoreType.DMA,
        ],
    )
    def k(color_hbm, order_hbm, out_hbm,
          ca0, ca1, oa0, oa1, cb0, cb1, ob0, ob1,
          b0, b1, orow_a, orow_b, isem, osem):
        wid = lax.axis_index("s") * info.num_cores + lax.axis_index("c")
        row0 = wid * rows_per_w
        iota = lax.iota(jnp.int32, L)
        zeros = jnp.zeros((L,), jnp.float32)
        last_lane = jnp.full((L,), L - 1, jnp.int32)

        def issue_in(r, c0, c1, o0, o1):
            pltpu.async_copy(color_hbm.at[0, r], c0, isem)
            pltpu.async_copy(color_hbm.at[1, r], c1, isem)
            pltpu.async_copy(order_hbm.at[0, r], o0, isem)
            pltpu.async_copy(order_hbm.at[1, r], o1, isem)

        def wait_in(r, c0, c1, o0, o1):
            pltpu.make_async_copy(color_hbm.at[0, r], c0, isem).wait()
            pltpu.make_async_copy(color_hbm.at[1, r], c1, isem).wait()
            pltpu.make_async_copy(order_hbm.at[0, r], o0, isem).wait()
            pltpu.make_async_copy(order_hbm.at[1, r], o1, isem).wait()

        @plsc.parallel_loop(0, n_bins_pad, step=L, unroll=4)
        def _(i):
            s = pl.ds(i, L)
            b0[s] = zeros
            b1[s] = zeros

        issue_in(row0, ca0, ca1, oa0, oa1)

        def process_row(r, c0, c1, o0, o1, orow):
            @plsc.parallel_loop(0, t_len, step=L, unroll=16)
            def _(i):
                s = pl.ds(i, L)
                plsc.addupdate_scatter(b0, [o0[s]], c0[s])
                plsc.addupdate_scatter(b1, [o1[s]], c1[s])

            def ff_chunk(s, cy0, cy1):
                v0 = b0[s]
                v1 = b1[s]
                g0 = _take16(v0, plsc.cummax(iota, mask=v0 != 0.0))
                g1 = _take16(v1, plsc.cummax(iota, mask=v1 != 0.0))
                f0 = jnp.where(g0 != 0.0, g0, cy0)
                f1 = jnp.where(g1 != 0.0, g1, cy1)
                b0[s] = zeros
                b1[s] = zeros
                orow[s] = f0 - f1
                return _take16(f0, last_lane), _take16(f1, last_lane)

            @plsc.parallel_loop(
                0, n_bins_pad, step=L, unroll=8, carry=(zeros, zeros)
            )
            def _(i, carry):
                cy0, cy1 = carry
                return ff_chunk(pl.ds(i, L), cy0, cy1)
            pltpu.async_copy(orow, out_hbm.at[r], osem)

        def pair_body(rp, _):
            ra = row0 + 2 * rp
            rb = ra + 1
            wait_in(ra, ca0, ca1, oa0, oa1)
            issue_in(rb, cb0, cb1, ob0, ob1)

            @pl.when(rp > 0)
            def _():
                pltpu.make_async_copy(orow_a, out_hbm.at[ra - 2], osem).wait()

            process_row(ra, ca0, ca1, oa0, oa1, orow_a)

            wait_in(rb, cb0, cb1, ob0, ob1)

            @pl.when(rp + 1 < rows_per_w // 2)
            def _():
                issue_in(rb + 1, ca0, ca1, oa0, oa1)

            @pl.when(rp > 0)
            def _():
                pltpu.make_async_copy(orow_b, out_hbm.at[rb - 2], osem).wait()

            process_row(rb, cb0, cb1, ob0, ob1, orow_b)
            return 0

        lax.fori_loop(0, rows_per_w // 2, pair_body, 0)

        last = row0 + rows_per_w - 1
        pltpu.make_async_copy(orow_a, out_hbm.at[last - 1], osem).wait()
        pltpu.make_async_copy(orow_b, out_hbm.at[last], osem).wait()

    return k(color, order)


def kernel(color, Ns, order):
    n_bands = color.shape[0]
    bsz = color.shape[1]
    ns_bands, ns_rows = Ns.shape
    n_bins = ns_rows * ns_bands * (ns_bands - 1) // 2 + ns_bands * (ns_rows - 1)
    n_bins_pad = (n_bins + 6 * L - 1) // (6 * L) * (6 * L)

    out = _mean_color_sc(color, order.astype(jnp.int32), n_bins_pad)
    return out[:, :n_bins].reshape(bsz, n_bins, 1)
